# baseline (speedup 1.0000x reference)
import jax
from jax import lax
import jax.numpy as jnp
from jax.experimental import pallas as pl
from jax.experimental.pallas import tpu as pltpu
from jax.experimental.pallas import tpu_sc as plsc

_N_NODES = 100000
_DIM = 128
_NUM_SPECIES = 128
_RB = 25088
_NB = 4


def _tc_lookup(node_specie, embeddings):
    idxp = jnp.pad(node_specie, (0, _NB * _RB - _N_NODES)).reshape(
        _NB, 1, _RB
    )

    def body(i_ref, w_ref, o_ref):
        ids = i_ref[0, 0, :]
        onehot = (
            ids[:, None]
            == lax.broadcasted_iota(jnp.int32, (_RB, _NUM_SPECIES), 1)
        ).astype(jnp.bfloat16)
        w = w_ref[...]
        w_hi = w.astype(jnp.bfloat16)
        r1 = w - w_hi.astype(jnp.float32)
        w_mid = r1.astype(jnp.bfloat16)
        acc = jnp.dot(onehot, w_hi, preferred_element_type=jnp.float32)
        acc = acc + jnp.dot(onehot, w_mid, preferred_element_type=jnp.float32)
        o_ref[...] = acc

    return pl.pallas_call(
        body,
        grid=(_NB,),
        in_specs=[
            pl.BlockSpec((1, 1, _RB), lambda i: (i, 0, 0)),
            pl.BlockSpec((_NUM_SPECIES, _DIM), lambda i: (0, 0)),
        ],
        out_specs=pl.BlockSpec((_RB, _DIM), lambda i: (i, 0)),
        out_shape=jax.ShapeDtypeStruct((_N_NODES, _DIM), jnp.float32),
    )(idxp, embeddings)


def kernel(node_specie, embeddings):
    return _tc_lookup(node_specie, embeddings)
